# baseline (speedup 1.0000x reference)
import functools

import jax
import jax.numpy as jnp
from jax import lax
from jax.experimental import pallas as pl
from jax.experimental.pallas import tpu as pltpu
from jax.experimental.pallas import tpu_sc as plsc

EMBEDDING_DIM = 128
MAX_LEAD_TIME = 72
BATCH = 16384
LANES = 16
IDX_CHUNK = 128


def kernel(lead_times, pe):
    info = plsc.get_sparse_core_info()
    num_cores, num_subcores = info.num_cores, info.num_subcores
    num_workers = num_cores * num_subcores
    b_per_w = BATCH // num_workers
    n_chunks = b_per_w // IDX_CHUNK

    mesh = plsc.VectorSubcoreMesh(core_axis_name="c", subcore_axis_name="s")

    @functools.partial(
        pl.kernel,
        mesh=mesh,
        out_type=jax.ShapeDtypeStruct((BATCH, EMBEDDING_DIM), jnp.float32),
        scratch_types=[
            pltpu.VMEM((b_per_w,), jnp.int32),
            pltpu.VMEM((b_per_w, EMBEDDING_DIM), jnp.float32),
            pltpu.SemaphoreType.DMA,
            pltpu.SemaphoreType.DMA,
            pltpu.SemaphoreType.DMA,
        ],
    )
    def emb_kernel(lt_hbm, pe_hbm, out_hbm, idx_v, rows_v, sem_a, sem_b, sem_out):
        wid = lax.axis_index("s") * num_cores + lax.axis_index("c")
        base = wid * b_per_w
        pltpu.sync_copy(lt_hbm.at[pl.ds(base, b_per_w)], idx_v)

        def clip_chunk(c):
            for i in range(IDX_CHUNK // LANES):
                o = c * IDX_CHUNK + i * LANES
                v = idx_v[pl.ds(o, LANES)]
                idx_v[pl.ds(o, LANES)] = jnp.minimum(
                    jnp.maximum(v, 0), MAX_LEAD_TIME
                )

        gsems = [sem_a, sem_b]

        def fire_gather(c):
            return pltpu.async_copy(
                pe_hbm.at[idx_v.at[pl.ds(c * IDX_CHUNK, IDX_CHUNK)]],
                rows_v.at[pl.ds(c * IDX_CHUNK, IDX_CHUNK)],
                gsems[c % 2],
            )

        gathers = {}
        for c in range(min(2, n_chunks)):
            clip_chunk(c)
            gathers[c] = fire_gather(c)
        for c in range(2, n_chunks):
            clip_chunk(c)

        outs = []
        for c in range(n_chunks):
            gathers[c].wait()
            outs.append(
                pltpu.async_copy(
                    rows_v.at[pl.ds(c * IDX_CHUNK, IDX_CHUNK)],
                    out_hbm.at[pl.ds(base + c * IDX_CHUNK, IDX_CHUNK)],
                    sem_out,
                )
            )
            if c + 2 < n_chunks:
                gathers[c + 2] = fire_gather(c + 2)
        for o in outs:
            o.wait()

    if lead_times.dtype != jnp.int32:
        lead_times = lead_times.astype(jnp.int32)
    return emb_kernel(lead_times, pe)


# speedup vs baseline: 1.1436x; 1.1436x over previous
import functools

import jax
import jax.numpy as jnp
from jax import lax
from jax.experimental import pallas as pl
from jax.experimental.pallas import tpu as pltpu
from jax.experimental.pallas import tpu_sc as plsc

EMBEDDING_DIM = 128
MAX_LEAD_TIME = 72
BATCH = 16384
LANES = 16
OUT_CHUNK = 128
UNROLL = 4


def kernel(lead_times, pe):
    info = plsc.get_sparse_core_info()
    num_cores, num_subcores = info.num_cores, info.num_subcores
    num_workers = num_cores * num_subcores
    b_per_w = BATCH // num_workers
    n_chunks = b_per_w // OUT_CHUNK
    vregs_per_row = EMBEDDING_DIM // LANES

    mesh = plsc.VectorSubcoreMesh(core_axis_name="c", subcore_axis_name="s")

    @functools.partial(
        pl.kernel,
        mesh=mesh,
        out_type=jax.ShapeDtypeStruct((BATCH, EMBEDDING_DIM), jnp.float32),
        scratch_types=[
            pltpu.VMEM((MAX_LEAD_TIME + 1, EMBEDDING_DIM), jnp.float32),
            pltpu.VMEM((b_per_w, EMBEDDING_DIM), jnp.float32),
            pltpu.VMEM((b_per_w,), jnp.int32),
            pltpu.SemaphoreType.DMA,
            pltpu.SemaphoreType.DMA,
        ],
    )
    def emb_kernel(
        lt_hbm, pe_hbm, out_hbm, pe_v, rows_v, idx_v, sem_in, sem_out
    ):
        wid = lax.axis_index("s") * num_cores + lax.axis_index("c")
        base = wid * b_per_w
        cp_tab = pltpu.async_copy(pe_hbm, pe_v, sem_in)
        pltpu.sync_copy(lt_hbm.at[pl.ds(base, b_per_w)], idx_v)
        cp_tab.wait()

        def do_group(g):
            b0 = g * LANES
            v_idx = idx_v[pl.ds(b0, LANES)]
            v_idx = jnp.minimum(jnp.maximum(v_idx, 0), MAX_LEAD_TIME)
            for u in range(LANES):
                r = v_idx[u]
                for j in range(vregs_per_row):
                    rows_v[b0 + u, pl.ds(j * LANES, LANES)] = pe_v[
                        r, pl.ds(j * LANES, LANES)
                    ]

        groups_per_chunk = OUT_CHUNK // LANES
        outs = []
        for c in range(n_chunks):

            @plsc.parallel_loop(
                c * groups_per_chunk, (c + 1) * groups_per_chunk, unroll=2
            )
            def _(g):
                do_group(g)

            outs.append(
                pltpu.async_copy(
                    rows_v.at[pl.ds(c * OUT_CHUNK, OUT_CHUNK)],
                    out_hbm.at[pl.ds(base + c * OUT_CHUNK, OUT_CHUNK)],
                    sem_out,
                )
            )
        for o in outs:
            o.wait()

    if lead_times.dtype != jnp.int32:
        lead_times = lead_times.astype(jnp.int32)
    return emb_kernel(lead_times, pe)


# speedup vs baseline: 1.4687x; 1.2843x over previous
import functools

import jax
import jax.numpy as jnp
from jax import lax
from jax.experimental import pallas as pl
from jax.experimental.pallas import tpu as pltpu
from jax.experimental.pallas import tpu_sc as plsc

EMBEDDING_DIM = 128
MAX_LEAD_TIME = 72
BATCH = 16384
LANES = 16
OUT_CHUNK = 128
UNROLL = 4


def kernel(lead_times, pe):
    info = plsc.get_sparse_core_info()
    num_cores, num_subcores = info.num_cores, info.num_subcores
    num_workers = num_cores * num_subcores
    b_per_w = BATCH // num_workers
    n_chunks = b_per_w // OUT_CHUNK
    vregs_per_row = EMBEDDING_DIM // LANES

    mesh = plsc.VectorSubcoreMesh(core_axis_name="c", subcore_axis_name="s")

    @functools.partial(
        pl.kernel,
        mesh=mesh,
        out_type=jax.ShapeDtypeStruct((BATCH, EMBEDDING_DIM), jnp.float32),
        scratch_types=[
            pltpu.VMEM((MAX_LEAD_TIME + 1, EMBEDDING_DIM), jnp.float32),
            pltpu.VMEM((b_per_w, EMBEDDING_DIM), jnp.float32),
            pltpu.VMEM((b_per_w,), jnp.int32),
            pltpu.SemaphoreType.DMA,
            pltpu.SemaphoreType.DMA,
        ],
    )
    def emb_kernel(
        lt_hbm, pe_hbm, out_hbm, pe_v, rows_v, idx_v, sem_in, sem_out
    ):
        wid = lax.axis_index("s") * num_cores + lax.axis_index("c")
        base = wid * b_per_w
        cp_tab = pltpu.async_copy(pe_hbm, pe_v, sem_in)
        pltpu.sync_copy(lt_hbm.at[pl.ds(base, b_per_w)], idx_v)
        cp_tab.wait()

        def do_group(b0):
            v_idx = idx_v[pl.ds(b0, LANES)]
            v_idx = jnp.minimum(jnp.maximum(v_idx, 0), MAX_LEAD_TIME)
            for u in range(LANES):
                r = v_idx[u]
                for j in range(vregs_per_row):
                    rows_v[b0 + u, pl.ds(j * LANES, LANES)] = pe_v[
                        r, pl.ds(j * LANES, LANES)
                    ]

        groups_per_chunk = OUT_CHUNK // LANES

        def chunk_body(c, carry):
            @plsc.parallel_loop(0, groups_per_chunk)
            def _(g):
                do_group(c * OUT_CHUNK + g * LANES)

            pltpu.async_copy(
                rows_v.at[pl.ds(c * OUT_CHUNK, OUT_CHUNK)],
                out_hbm.at[pl.ds(base + c * OUT_CHUNK, OUT_CHUNK)],
                sem_out,
            )
            return carry

        lax.fori_loop(0, n_chunks, chunk_body, 0)
        pltpu.make_async_copy(
            out_hbm.at[pl.ds(base, b_per_w)], rows_v, sem_out
        ).wait()

    if lead_times.dtype != jnp.int32:
        lead_times = lead_times.astype(jnp.int32)
    return emb_kernel(lead_times, pe)
